# baseline (speedup 1.0000x reference)
import functools
from math import sqrt

import jax
import jax.numpy as jnp
from jax import lax
from jax.experimental import pallas as pl
from jax.experimental.pallas import tpu as pltpu
from jax.experimental.pallas import tpu_sc as plsc

_NC = 2
_NS = 16
_NW = _NC * _NS
_LANES = 16
_NBUF = 2


def _sc_body(xf, pef, out, xin, pein, xout, sx0, sx1, sp0, sp1, so0, so1, *,
             base_row, rows_per_worker, chunk_rows, d, seq, scale):
    w = lax.axis_index("s") * _NC + lax.axis_index("c")
    row0 = w * rows_per_worker
    pe_row0 = lax.rem(base_row + row0, seq)
    ce = chunk_rows * d
    nchunks = rows_per_worker // chunk_rows
    sx = (sx0, sx1)
    sp = (sp0, sp1)
    so = (so0, so1)

    def in_copies(c, b):
        off = (base_row * d) + (row0 + c * chunk_rows) * d
        pe_off = (pe_row0 + c * chunk_rows) * d
        cx = pltpu.make_async_copy(xf.at[pl.ds(off, ce)], xin.at[b], sx[b])
        cp = pltpu.make_async_copy(pef.at[pl.ds(pe_off, ce)], pein.at[b],
                                   sp[b])
        return cx, cp

    def out_copy(c, b):
        off = (row0 + c * chunk_rows) * d
        return pltpu.make_async_copy(xout.at[b], out.at[pl.ds(off, ce)],
                                     so[b])

    for b in range(_NBUF):
        cx, cp = in_copies(b, b)
        cx.start()
        cp.start()

    def chunk(c0, _):
        for b in range(_NBUF):
            c = c0 * _NBUF + b
            cx, cp = in_copies(c, b)
            cx.wait()
            cp.wait()

            @pl.when(c >= _NBUF)
            def _():
                out_copy(c - _NBUF, b).wait()

            @plsc.parallel_loop(0, ce, step=_LANES, unroll=8)
            def _(i):
                sl = pl.ds(i, _LANES)
                xout[b, sl] = xin[b, sl] * scale + pein[b, sl]

            out_copy(c, b).start()

            @pl.when(c + _NBUF < nchunks)
            def _():
                ncx, ncp = in_copies(c + _NBUF, b)
                ncx.start()
                ncp.start()
        return 0

    lax.fori_loop(0, nchunks // _NBUF, chunk, 0)

    for b in range(_NBUF):
        out_copy(nchunks - _NBUF + b, b).wait()


def _sc_shard(xf, pe, base_row, rows, chunk_rows=16):
    seq, d = pe.shape
    scale = sqrt(float(d))
    n = rows * d
    rows_per_worker = rows // _NW
    ce = chunk_rows * d

    sc_fn = pl.kernel(
        functools.partial(
            _sc_body,
            base_row=base_row,
            rows_per_worker=rows_per_worker,
            chunk_rows=chunk_rows,
            d=d,
            seq=seq,
            scale=scale,
        ),
        out_type=jax.ShapeDtypeStruct((n,), jnp.float32),
        mesh=plsc.VectorSubcoreMesh(core_axis_name="c", subcore_axis_name="s"),
        scratch_types=[
            pltpu.VMEM((_NBUF, ce), jnp.float32),
            pltpu.VMEM((_NBUF, ce), jnp.float32),
            pltpu.VMEM((_NBUF, ce), jnp.float32),
            pltpu.SemaphoreType.DMA,
            pltpu.SemaphoreType.DMA,
            pltpu.SemaphoreType.DMA,
            pltpu.SemaphoreType.DMA,
            pltpu.SemaphoreType.DMA,
            pltpu.SemaphoreType.DMA,
        ],
    )
    return sc_fn(xf, pe.reshape(seq * d)).reshape(rows, d)


def _tc_kernel(x_ref, pe_ref, out_ref, *, scale):
    out_ref[...] = x_ref[...] * scale + pe_ref[...]


def _tc_shard(x, pe, n_batch, blk_s=2048):
    batch, seq, d = x.shape
    scale = sqrt(float(d))
    grid = (seq // blk_s, n_batch)
    return pl.pallas_call(
        functools.partial(_tc_kernel, scale=scale),
        grid=grid,
        in_specs=[
            pl.BlockSpec((1, blk_s, d), lambda i, j: (j, i, 0)),
            pl.BlockSpec((blk_s, d), lambda i, j: (i, 0)),
        ],
        out_specs=pl.BlockSpec((1, blk_s, d), lambda i, j: (j, i, 0)),
        out_shape=jax.ShapeDtypeStruct((n_batch, seq, d), x.dtype),
        compiler_params=pltpu.CompilerParams(
            dimension_semantics=("parallel", "parallel"),
            vmem_limit_bytes=128 * 1024 * 1024,
        ),
    )(x, pe)


def kernel(x, emb_table):
    batch, seq, d = x.shape
    pe = emb_table[:seq]
    n_tc = batch - 1

    out_sc = _sc_shard(x.reshape(batch * seq * d), pe, n_tc * seq, seq)
    out_tc = _tc_shard(x, pe, n_tc)
    return jnp.concatenate([out_tc, out_sc[None]], axis=0)


# speedup vs baseline: 3.9223x; 3.9223x over previous
import functools
from math import sqrt

import jax
import jax.numpy as jnp
from jax.experimental import pallas as pl
from jax.experimental.pallas import tpu as pltpu


def _pe_add_kernel(x_ref, pe_ref, out_ref, *, scale):
    out_ref[...] = x_ref[...] * scale + pe_ref[...]


def kernel(x, emb_table):
    batch, seq, d = x.shape
    scale = sqrt(float(d))
    blk_s = 2048
    grid = (seq // blk_s, batch)

    return pl.pallas_call(
        functools.partial(_pe_add_kernel, scale=scale),
        grid=grid,
        in_specs=[
            pl.BlockSpec((1, blk_s, d), lambda i, j: (j, i, 0)),
            pl.BlockSpec((blk_s, d), lambda i, j: (i, 0)),
        ],
        out_specs=pl.BlockSpec((1, blk_s, d), lambda i, j: (j, i, 0)),
        out_shape=jax.ShapeDtypeStruct((batch, seq, d), x.dtype),
        compiler_params=pltpu.CompilerParams(
            dimension_semantics=("parallel", "parallel"),
            vmem_limit_bytes=128 * 1024 * 1024,
        ),
    )(x, emb_table[:seq])
